# baseline (speedup 1.0000x reference)
import jax
import jax.numpy as jnp
from jax.experimental import pallas as pl
from jax.experimental.pallas import tpu as pltpu

_S = 4
_V = 32
_BS = 8
_D = 1024
_P = _V * _BS
_R = _V * _P
_HI = jax.lax.Precision.HIGHEST


def _fused_kernel(fg_ref, w1_ref, b1_ref, ds_ref, dt1_ref, dt2_ref, dt3_ref,
                  wt1_ref, bt1_ref, wt2_ref, bt2_ref, wt3_ref, bt3_ref,
                  out_ref, w1_vmem, w1_sem):
    w1_copy = pltpu.make_async_copy(w1_ref, w1_vmem, w1_sem)
    w1_copy.start()

    row = jax.lax.broadcasted_iota(jnp.int32, (_V, _D), 0)
    col = jax.lax.broadcasted_iota(jnp.int32, (_V, _D), 1)
    blk = (col // _V == row).astype(jnp.float32)

    ones_row = jnp.full((1, _D), 1.0, jnp.float32)

    def s2(dref):
        d = dref[...]
        cn = jnp.sqrt(jnp.dot(ones_row, d * d,
                              preferred_element_type=jnp.float32))
        bs = jnp.dot(blk, d, preferred_element_type=jnp.float32)
        return bs / jnp.maximum(cn, 1e-12)

    s2_s = s2(ds_ref)
    s2_t1 = s2(dt1_ref)
    s2_t2 = s2(dt2_ref)
    s2_t3 = s2(dt3_ref)

    w1_copy.wait()
    fm0 = (
        jnp.dot(fg_ref[...], w1_vmem[...],
                preferred_element_type=jnp.float32) + b1_ref[...]
    )
    u = jnp.swapaxes(fm0.transpose(1, 0).reshape(_V, _V, _BS), 1, 2)
    u = u.reshape(_P, _V)

    def pairs(x):
        c = x.shape[-1]
        x4 = jnp.broadcast_to(x.reshape(1, _V, _BS, c), (_V, _V, _BS, c))
        return x4.reshape(_R, c)

    def centers(x):
        c = x.shape[-1]
        x4 = jnp.broadcast_to(x.reshape(_V, 1, _BS, c), (_V, _V, _BS, c))
        return x4.reshape(_R, c)

    dif = pairs(u) - centers(u)
    d2r = jnp.sum(dif * dif, axis=1, keepdims=True) * float(_V)
    inv_norm = jax.lax.rsqrt(jnp.maximum(d2r, 1e-24))

    d4 = d2r.reshape(_V, _V, _BS, 1)
    dmin = jnp.min(d4, axis=1, keepdims=True)
    ji4 = jax.lax.broadcasted_iota(jnp.int32, (_V, _V, _BS, 1), 1)
    near = jnp.min(jnp.where(d4 <= jnp.broadcast_to(dmin, d4.shape),
                             ji4, 2 ** 30),
                   axis=1, keepdims=True)
    exclude = ji4 == jnp.broadcast_to(near, d4.shape)
    negmask = jnp.where(exclude, -jnp.inf, 0.0).reshape(_R, 1)
    inv_norm_z = jnp.where(exclude.reshape(_R, 1), 0.0, inv_norm)

    def combine(s2m, oc, support):
        g = jnp.dot(u, s2m,
                    preferred_element_type=jnp.float32)
        if support is None:
            th = jax.nn.relu((pairs(g) - centers(g)) * inv_norm_z)
        else:
            th = jax.nn.relu((pairs(g) - centers(g)) * inv_norm)
            th = th * pairs(support)
            th = th + negmask
        m = jnp.max(th.reshape(_V, _V, _BS, _S * oc), axis=1)
        mm = m.reshape(_P, _S * oc)
        acc = mm[:, :oc]
        for s in range(1, _S):
            acc = acc + mm[:, s * oc:(s + 1) * oc]
        return acc

    fm1 = jax.nn.relu(combine(s2_s, 32, None))
    fo = jnp.dot(fm1, wt1_ref[...],
                 preferred_element_type=jnp.float32) + bt1_ref[...]
    fm2 = jax.nn.relu(fo[:, :32] + combine(s2_t1, 32, fo[:, 32:]))
    fo = jnp.dot(fm2, wt2_ref[...],
                 preferred_element_type=jnp.float32) + bt2_ref[...]
    fm4 = jax.nn.relu(fo[:, :16] + combine(s2_t2, 16, fo[:, 16:]))
    fo = jnp.dot(fm4, wt3_ref[...],
                 preferred_element_type=jnp.float32) + bt3_ref[...]
    res = fo[:, :3] + combine(s2_t3, 3, fo[:, 3:])
    out_ref[...] = jnp.swapaxes(res.reshape(_V, _BS, 3), 0, 1)


def kernel(feature_global, W1, b1, dir_s, w_t1, b_t1, dir_t1,
           w_t2, b_t2, dir_t2, w_t3, b_t3, dir_t3):
    f32 = jnp.float32
    vmem = pltpu.MemorySpace.VMEM
    return pl.pallas_call(
        _fused_kernel,
        in_specs=[
            pl.BlockSpec(memory_space=vmem),
            pl.BlockSpec(memory_space=pltpu.MemorySpace.HBM),
            pl.BlockSpec(memory_space=vmem),
            pl.BlockSpec(memory_space=vmem),
            pl.BlockSpec(memory_space=vmem),
            pl.BlockSpec(memory_space=vmem),
            pl.BlockSpec(memory_space=vmem),
            pl.BlockSpec(memory_space=vmem),
            pl.BlockSpec(memory_space=vmem),
            pl.BlockSpec(memory_space=vmem),
            pl.BlockSpec(memory_space=vmem),
            pl.BlockSpec(memory_space=vmem),
            pl.BlockSpec(memory_space=vmem),
        ],
        scratch_shapes=[
            pltpu.MemorySpace.VMEM((512, _D), f32),
            pltpu.SemaphoreType.DMA,
        ],
        out_shape=jax.ShapeDtypeStruct((_BS, _V, 3), f32),
    )(feature_global, W1, b1.reshape(1, _D), dir_s, dir_t1, dir_t2, dir_t3,
      w_t1, b_t1.reshape(1, 160), w_t2, b_t2.reshape(1, 80),
      w_t3, b_t3.reshape(1, 15))


# speedup vs baseline: 1.0416x; 1.0416x over previous
import jax
import jax.numpy as jnp
from jax.experimental import pallas as pl
from jax.experimental.pallas import tpu as pltpu

_S = 4
_V = 32
_BS = 8
_D = 1024
_P = _V * _BS
_R = _V * _P
_HI = jax.lax.Precision.HIGHEST


def _fused_kernel(fg_ref, w1_ref, b1_ref, ds_ref, dt1_ref, dt2_ref, dt3_ref,
                  wt1_ref, bt1_ref, wt2_ref, bt2_ref, wt3_ref, bt3_ref,
                  out_ref):
    row = jax.lax.broadcasted_iota(jnp.int32, (_V, _D), 0)
    col = jax.lax.broadcasted_iota(jnp.int32, (_V, _D), 1)
    blk = (col // _V == row).astype(jnp.float32)

    ones_row = jnp.full((1, _D), 1.0, jnp.float32)

    def s2(dref):
        d = dref[...]
        cn = jnp.sqrt(jnp.dot(ones_row, d * d,
                              preferred_element_type=jnp.float32))
        bs = jnp.dot(blk, d, preferred_element_type=jnp.float32)
        return bs / jnp.maximum(cn, 1e-12)

    s2_s = s2(ds_ref)
    s2_t1 = s2(dt1_ref)
    s2_t2 = s2(dt2_ref)
    s2_t3 = s2(dt3_ref)

    fm0 = (
        jnp.dot(fg_ref[...], w1_ref[...],
                preferred_element_type=jnp.float32) + b1_ref[...]
    )
    u = jnp.swapaxes(fm0.transpose(1, 0).reshape(_V, _V, _BS), 1, 2)
    u = u.reshape(_P, _V)

    def pairs(x):
        c = x.shape[-1]
        x4 = jnp.broadcast_to(x.reshape(1, _V, _BS, c), (_V, _V, _BS, c))
        return x4.reshape(_R, c)

    def centers(x):
        c = x.shape[-1]
        x4 = jnp.broadcast_to(x.reshape(_V, 1, _BS, c), (_V, _V, _BS, c))
        return x4.reshape(_R, c)

    dif = pairs(u) - centers(u)
    d2r = jnp.sum(dif * dif, axis=1, keepdims=True) * float(_V)
    inv_norm = jax.lax.rsqrt(jnp.maximum(d2r, 1e-24))

    d4 = d2r.reshape(_V, _V, _BS, 1)
    dmin = jnp.min(d4, axis=1, keepdims=True)
    ji4 = jax.lax.broadcasted_iota(jnp.int32, (_V, _V, _BS, 1), 1)
    near = jnp.min(jnp.where(d4 <= jnp.broadcast_to(dmin, d4.shape),
                             ji4, 2 ** 30),
                   axis=1, keepdims=True)
    exclude = ji4 == jnp.broadcast_to(near, d4.shape)
    negmask = jnp.where(exclude, -jnp.inf, 0.0).reshape(_R, 1)

    s2_all = jnp.concatenate([s2_s, s2_t1, s2_t2, s2_t3], axis=1)
    g_all = jnp.dot(u, s2_all,
                    preferred_element_type=jnp.float32)
    th_all = jax.nn.relu((pairs(g_all) - centers(g_all)) * inv_norm)

    def combine(off, oc, support):
        th = th_all[:, off:off + _S * oc]
        if support is not None:
            th = th * pairs(support)
        th = th + negmask
        m = jnp.max(th.reshape(_V, _V, _BS, _S * oc), axis=1)
        mm = m.reshape(_P, _S * oc)
        acc = mm[:, :oc]
        for s in range(1, _S):
            acc = acc + mm[:, s * oc:(s + 1) * oc]
        return acc

    fm1 = jax.nn.relu(combine(0, 32, None))
    fo = jnp.dot(fm1, wt1_ref[...],
                 preferred_element_type=jnp.float32) + bt1_ref[...]
    fm2 = jax.nn.relu(fo[:, :32] + combine(128, 32, fo[:, 32:]))
    fo = jnp.dot(fm2, wt2_ref[...],
                 preferred_element_type=jnp.float32) + bt2_ref[...]
    fm4 = jax.nn.relu(fo[:, :16] + combine(256, 16, fo[:, 16:]))
    fo = jnp.dot(fm4, wt3_ref[...],
                 preferred_element_type=jnp.float32) + bt3_ref[...]
    res = fo[:, :3] + combine(320, 3, fo[:, 3:])
    out_ref[...] = jnp.swapaxes(res.reshape(_V, _BS, 3), 0, 1)


def kernel(feature_global, W1, b1, dir_s, w_t1, b_t1, dir_t1,
           w_t2, b_t2, dir_t2, w_t3, b_t3, dir_t3):
    f32 = jnp.float32
    return pl.pallas_call(
        _fused_kernel,
        out_shape=jax.ShapeDtypeStruct((_BS, _V, 3), f32),
    )(feature_global, W1, b1.reshape(1, _D), dir_s, dir_t1, dir_t2, dir_t3,
      w_t1, b_t1.reshape(1, 160), w_t2, b_t2.reshape(1, 80),
      w_t3, b_t3.reshape(1, 15))
